# baseline (speedup 1.0000x reference)
import functools

import jax
import jax.numpy as jnp
from jax import lax
from jax.experimental import pallas as pl
from jax.experimental.pallas import tpu as pltpu
from jax.experimental.pallas import tpu_sc as plsc

K_NB = 16
_BIG = 1e9
_L = 16
_CHK = 32
_PAD = 128


def _prep_kernel(x_ref, xe_ref):
    x = x_ref[0]
    xt = jnp.transpose(x)
    nrm = jnp.sqrt(jnp.sum(xt * xt, axis=1, keepdims=True))
    xn = xt * (1.0 / jnp.maximum(nrm, 1e-12))
    sq = jnp.sum(xn * xn, axis=1, keepdims=True)
    g = lax.dot_general(xn, xn, (((1,), (1,)), ((), ())),
                        preferred_element_type=jnp.float32)
    d2 = sq + jnp.transpose(sq) - 2.0 * g
    d2 = jnp.maximum(d2, 0.0)
    N = d2.shape[0]
    rowid = lax.broadcasted_iota(jnp.int32, (N, N), 0)
    colid = lax.broadcasted_iota(jnp.int32, (N, N), 1)
    d2 = jnp.where(rowid == colid, _BIG, d2)
    rows = []
    for _ in range(K_NB):
        m = jnp.min(d2, axis=0, keepdims=True)
        cand = jnp.where(d2 == m, rowid, N)
        first = jnp.min(cand, axis=0, keepdims=True)
        rows.append(first)
        d2 = jnp.where(rowid == first, _BIG, d2)
    nnf = jnp.transpose(
        jnp.concatenate(rows, axis=0).astype(jnp.float32))
    pad = jnp.zeros((N, _PAD - K_NB), jnp.float32)
    xe_ref[0] = jnp.concatenate([xn, nnf, pad], axis=1)


def _knn_sc_kernel(xe_hbm, md_hbm, xnt_v, idx_v, md_v, *, n, c):
    nc = 2
    b = lax.axis_index("s") * nc + lax.axis_index("c")
    pltpu.sync_copy(xe_hbm.at[b, :, pl.ds(0, c)], xnt_v)
    pltpu.sync_copy(xe_hbm.at[b, :, pl.ds(c, K_NB)], idx_v)
    ncc = c // _L
    nh = ncc // 3

    def chunk_body(ch, carry0):
        def row_body(ip, carry1):
            il0 = 2 * ip
            il1 = il0 + 1
            i0 = ch * _CHK + il0
            i1 = i0 + 1
            ix0 = idx_v[i0, :].astype(jnp.int32)
            ix1 = idx_v[i1, :].astype(jnp.int32)
            for h in range(3):
                lo = h * nh
                mx0 = [xnt_v[i0, pl.ds((lo + cc) * _L, _L)]
                       for cc in range(nh)]
                mn0 = list(mx0)
                mx1 = [xnt_v[i1, pl.ds((lo + cc) * _L, _L)]
                       for cc in range(nh)]
                mn1 = list(mx1)
                for t in range(K_NB):
                    j0 = ix0[t]
                    j1 = ix1[t]
                    for cc in range(nh):
                        nb0 = xnt_v[j0, pl.ds((lo + cc) * _L, _L)]
                        nb1 = xnt_v[j1, pl.ds((lo + cc) * _L, _L)]
                        mx0[cc] = jnp.maximum(mx0[cc], nb0)
                        mn0[cc] = jnp.minimum(mn0[cc], nb0)
                        mx1[cc] = jnp.maximum(mx1[cc], nb1)
                        mn1[cc] = jnp.minimum(mn1[cc], nb1)
                for cc in range(nh):
                    xi0 = xnt_v[i0, pl.ds((lo + cc) * _L, _L)]
                    xi1 = xnt_v[i1, pl.ds((lo + cc) * _L, _L)]
                    md_v[il0, pl.ds((lo + cc) * _L, _L)] = jnp.maximum(
                        mx0[cc] - xi0, xi0 - mn0[cc])
                    md_v[il1, pl.ds((lo + cc) * _L, _L)] = jnp.maximum(
                        mx1[cc] - xi1, xi1 - mn1[cc])
            return carry1

        lax.fori_loop(0, _CHK // 2, row_body, 0)
        pltpu.sync_copy(md_v, md_hbm.at[b, pl.ds(ch * _CHK, _CHK)])
        return carry0

    lax.fori_loop(0, n // _CHK, chunk_body, 0)


def _conv_kernel(we_ref, wo_ref, bias_ref, xe_ref, md_ref,
                 y_ref, s1_ref, s2_ref, *, c):
    b = pl.program_id(0)
    xnb = xe_ref[0][:, :c].astype(jnp.bfloat16)
    mdb = md_ref[0].astype(jnp.bfloat16)
    y = lax.dot_general(we_ref[...], xnb, (((1,), (1,)), ((), ())),
                        preferred_element_type=jnp.float32)
    y = y + lax.dot_general(wo_ref[...], mdb, (((1,), (1,)), ((), ())),
                            preferred_element_type=jnp.float32)
    y = y + bias_ref[...]
    y_ref[0] = y
    ps1 = jnp.sum(y, axis=1, keepdims=True)
    ps2 = jnp.sum(y * y, axis=1, keepdims=True)

    @pl.when(b == 0)
    def _():
        s1_ref[...] = ps1
        s2_ref[...] = ps2

    @pl.when(b != 0)
    def _():
        s1_ref[...] += ps1
        s2_ref[...] += ps2


def _bn_gelu_kernel(y_ref, s1_ref, s2_ref, gamma_ref, beta_ref, o_ref,
                    *, count):
    mean = s1_ref[...] * (1.0 / count)
    var = s2_ref[...] * (1.0 / count) - mean * mean
    scale = gamma_ref[...] * lax.rsqrt(var + 1e-5)
    shift = beta_ref[...] - mean * scale
    yn = y_ref[0] * scale + shift
    o_ref[0] = yn * 0.5 * (1.0 + lax.erf(yn * 0.7071067811865476))


def kernel(x, W, b, gamma, beta):
    B, C, N = x.shape
    O = W.shape[0]
    CE = C + _PAD
    we = W[:, 0::2].astype(jnp.bfloat16)
    wo = W[:, 1::2].astype(jnp.bfloat16)

    xe = pl.pallas_call(
        _prep_kernel,
        grid=(B,),
        in_specs=[pl.BlockSpec((1, C, N), lambda i: (i, 0, 0))],
        out_specs=pl.BlockSpec((1, N, CE), lambda i: (i, 0, 0)),
        out_shape=jax.ShapeDtypeStruct((B, N, CE), jnp.float32),
    )(x)

    md = pl.kernel(
        functools.partial(_knn_sc_kernel, n=N, c=C),
        mesh=plsc.VectorSubcoreMesh(core_axis_name="c", subcore_axis_name="s"),
        compiler_params=pltpu.CompilerParams(use_tc_tiling_on_sc=False),
        out_type=jax.ShapeDtypeStruct((B, N, C), jnp.float32),
        scratch_types=[
            pltpu.VMEM((N, C), jnp.float32),
            pltpu.VMEM((N, K_NB), jnp.float32),
            pltpu.VMEM((_CHK, C), jnp.float32),
        ],
    )(xe)

    y, s1, s2 = pl.pallas_call(
        functools.partial(_conv_kernel, c=C),
        grid=(B,),
        in_specs=[pl.BlockSpec((O, C), lambda i: (0, 0)),
                  pl.BlockSpec((O, C), lambda i: (0, 0)),
                  pl.BlockSpec((O, 1), lambda i: (0, 0)),
                  pl.BlockSpec((1, N, CE), lambda i: (i, 0, 0)),
                  pl.BlockSpec((1, N, C), lambda i: (i, 0, 0))],
        out_specs=[pl.BlockSpec((1, O, N), lambda i: (i, 0, 0)),
                   pl.BlockSpec((O, 1), lambda i: (0, 0)),
                   pl.BlockSpec((O, 1), lambda i: (0, 0))],
        out_shape=[jax.ShapeDtypeStruct((B, O, N), jnp.float32),
                   jax.ShapeDtypeStruct((O, 1), jnp.float32),
                   jax.ShapeDtypeStruct((O, 1), jnp.float32)],
    )(we, wo, b.reshape(O, 1), xe, md)

    out = pl.pallas_call(
        functools.partial(_bn_gelu_kernel, count=float(B * N)),
        grid=(B,),
        in_specs=[pl.BlockSpec((1, O, N), lambda i: (i, 0, 0)),
                  pl.BlockSpec((O, 1), lambda i: (0, 0)),
                  pl.BlockSpec((O, 1), lambda i: (0, 0)),
                  pl.BlockSpec((O, 1), lambda i: (0, 0)),
                  pl.BlockSpec((O, 1), lambda i: (0, 0))],
        out_specs=pl.BlockSpec((1, O, N), lambda i: (i, 0, 0)),
        out_shape=jax.ShapeDtypeStruct((B, O, N), jnp.float32),
    )(y, s1, s2, gamma.reshape(O, 1), beta.reshape(O, 1))

    return out.reshape(B, O, N, 1)


# speedup vs baseline: 1.5252x; 1.5252x over previous
import functools

import jax
import jax.numpy as jnp
from jax import lax
from jax.experimental import pallas as pl
from jax.experimental.pallas import tpu as pltpu
from jax.experimental.pallas import tpu_sc as plsc

K_NB = 16
_BIG = 1e9
_L = 16
_CHK = 64
_PAD = 128


def _prep_kernel(x_ref, xe_ref):
    x = x_ref[0]
    xt = jnp.transpose(x)
    nrm = jnp.sqrt(jnp.sum(xt * xt, axis=1, keepdims=True))
    xn = xt * (1.0 / jnp.maximum(nrm, 1e-12))
    sq = jnp.sum(xn * xn, axis=1, keepdims=True)
    g = lax.dot_general(xn, xn, (((1,), (1,)), ((), ())),
                        preferred_element_type=jnp.float32)
    d2 = sq + jnp.transpose(sq) - 2.0 * g
    d2 = jnp.maximum(d2, 0.0)
    N = d2.shape[0]
    rowid = lax.broadcasted_iota(jnp.int32, (N, N), 0)
    colid = lax.broadcasted_iota(jnp.int32, (N, N), 1)
    d2 = jnp.where(rowid == colid, _BIG, d2)
    rows = []
    for _ in range(K_NB):
        m = jnp.min(d2, axis=0, keepdims=True)
        cand = jnp.where(d2 == m, rowid, N)
        first = jnp.min(cand, axis=0, keepdims=True)
        rows.append(first)
        d2 = jnp.where(rowid == first, _BIG, d2)
    nnf = jnp.transpose(
        jnp.concatenate(rows, axis=0).astype(jnp.float32))
    pad = jnp.zeros((N, _PAD - K_NB), jnp.float32)
    xe_ref[0] = jnp.concatenate([xn, nnf, pad], axis=1)


def _knn_sc_kernel(xe_hbm, md_hbm, xnt_v, idx_v, md_v, *, n, c):
    nc = 2
    b = lax.axis_index("s") * nc + lax.axis_index("c")
    pltpu.sync_copy(xe_hbm.at[b, :, pl.ds(0, c)], xnt_v)
    pltpu.sync_copy(xe_hbm.at[b, :, pl.ds(c, K_NB)], idx_v)
    ncc = c // _L
    nh = ncc // 2

    def chunk_body(ch, carry0):
        def row_body(il, carry1):
            i = ch * _CHK + il
            idxv = idx_v[i, :].astype(jnp.int32)
            for h in range(2):
                lo = h * nh
                mx = [xnt_v[i, pl.ds((lo + cc) * _L, _L)] for cc in range(nh)]
                mn = list(mx)
                for t in range(K_NB):
                    jsc = idxv[t]
                    for cc in range(nh):
                        nb = xnt_v[jsc, pl.ds((lo + cc) * _L, _L)]
                        mx[cc] = jnp.maximum(mx[cc], nb)
                        mn[cc] = jnp.minimum(mn[cc], nb)
                for cc in range(nh):
                    xi = xnt_v[i, pl.ds((lo + cc) * _L, _L)]
                    md_v[il, pl.ds((lo + cc) * _L, _L)] = jnp.maximum(
                        mx[cc] - xi, xi - mn[cc])
            return carry1

        lax.fori_loop(0, _CHK, row_body, 0)
        pltpu.sync_copy(md_v, md_hbm.at[b, pl.ds(ch * _CHK, _CHK)])
        return carry0

    lax.fori_loop(0, n // _CHK, chunk_body, 0)


def _conv_kernel(we_ref, wo_ref, bias_ref, xe_ref, md_ref,
                 y_ref, s1_ref, s2_ref, *, c):
    b = pl.program_id(0)
    xnb = xe_ref[0][:, :c].astype(jnp.bfloat16)
    mdb = md_ref[0].astype(jnp.bfloat16)
    y = lax.dot_general(we_ref[...], xnb, (((1,), (1,)), ((), ())),
                        preferred_element_type=jnp.float32)
    y = y + lax.dot_general(wo_ref[...], mdb, (((1,), (1,)), ((), ())),
                            preferred_element_type=jnp.float32)
    y = y + bias_ref[...]
    y_ref[0] = y
    ps1 = jnp.sum(y, axis=1, keepdims=True)
    ps2 = jnp.sum(y * y, axis=1, keepdims=True)

    @pl.when(b == 0)
    def _():
        s1_ref[...] = ps1
        s2_ref[...] = ps2

    @pl.when(b != 0)
    def _():
        s1_ref[...] += ps1
        s2_ref[...] += ps2


def _bn_gelu_kernel(y_ref, s1_ref, s2_ref, gamma_ref, beta_ref, o_ref,
                    *, count):
    mean = s1_ref[...] * (1.0 / count)
    var = s2_ref[...] * (1.0 / count) - mean * mean
    scale = gamma_ref[...] * lax.rsqrt(var + 1e-5)
    shift = beta_ref[...] - mean * scale
    yn = y_ref[0] * scale + shift
    o_ref[0] = yn * 0.5 * (1.0 + lax.erf(yn * 0.7071067811865476))


def kernel(x, W, b, gamma, beta):
    B, C, N = x.shape
    O = W.shape[0]
    CE = C + _PAD
    we = W[:, 0::2].astype(jnp.bfloat16)
    wo = W[:, 1::2].astype(jnp.bfloat16)

    xe = pl.pallas_call(
        _prep_kernel,
        grid=(B,),
        in_specs=[pl.BlockSpec((1, C, N), lambda i: (i, 0, 0))],
        out_specs=pl.BlockSpec((1, N, CE), lambda i: (i, 0, 0)),
        out_shape=jax.ShapeDtypeStruct((B, N, CE), jnp.float32),
    )(x)

    md = pl.kernel(
        functools.partial(_knn_sc_kernel, n=N, c=C),
        mesh=plsc.VectorSubcoreMesh(core_axis_name="c", subcore_axis_name="s"),
        compiler_params=pltpu.CompilerParams(use_tc_tiling_on_sc=False),
        out_type=jax.ShapeDtypeStruct((B, N, C), jnp.float32),
        scratch_types=[
            pltpu.VMEM((N, C), jnp.float32),
            pltpu.VMEM((N, K_NB), jnp.float32),
            pltpu.VMEM((_CHK, C), jnp.float32),
        ],
    )(xe)

    y, s1, s2 = pl.pallas_call(
        functools.partial(_conv_kernel, c=C),
        grid=(B,),
        in_specs=[pl.BlockSpec((O, C), lambda i: (0, 0)),
                  pl.BlockSpec((O, C), lambda i: (0, 0)),
                  pl.BlockSpec((O, 1), lambda i: (0, 0)),
                  pl.BlockSpec((1, N, CE), lambda i: (i, 0, 0)),
                  pl.BlockSpec((1, N, C), lambda i: (i, 0, 0))],
        out_specs=[pl.BlockSpec((1, O, N), lambda i: (i, 0, 0)),
                   pl.BlockSpec((O, 1), lambda i: (0, 0)),
                   pl.BlockSpec((O, 1), lambda i: (0, 0))],
        out_shape=[jax.ShapeDtypeStruct((B, O, N), jnp.float32),
                   jax.ShapeDtypeStruct((O, 1), jnp.float32),
                   jax.ShapeDtypeStruct((O, 1), jnp.float32)],
    )(we, wo, b.reshape(O, 1), xe, md)

    out = pl.pallas_call(
        functools.partial(_bn_gelu_kernel, count=float(B * N)),
        grid=(B,),
        in_specs=[pl.BlockSpec((1, O, N), lambda i: (i, 0, 0)),
                  pl.BlockSpec((O, 1), lambda i: (0, 0)),
                  pl.BlockSpec((O, 1), lambda i: (0, 0)),
                  pl.BlockSpec((O, 1), lambda i: (0, 0)),
                  pl.BlockSpec((O, 1), lambda i: (0, 0))],
        out_specs=pl.BlockSpec((1, O, N), lambda i: (i, 0, 0)),
        out_shape=jax.ShapeDtypeStruct((B, O, N), jnp.float32),
    )(y, s1, s2, gamma.reshape(O, 1), beta.reshape(O, 1))

    return out.reshape(B, O, N, 1)
